# baseline (speedup 1.0000x reference)
import functools

import jax
import jax.numpy as jnp
from jax import lax
from jax.experimental import pallas as pl
from jax.experimental.pallas import tpu as pltpu
from jax.experimental.pallas import tpu_sc as plsc

_NC = 2
_NS = 16
_NW = _NC * _NS
_CH = 128


@functools.lru_cache(maxsize=None)
def _build(n, d, n_ch):
    mesh = plsc.VectorSubcoreMesh(core_axis_name="c", subcore_axis_name="s")

    @functools.partial(
        pl.kernel,
        mesh=mesh,
        out_type=jax.ShapeDtypeStruct((n, d), jnp.float32),
        scratch_types=[
            pltpu.VMEM((n_ch, _CH), jnp.int32),
            pltpu.VMEM((_CH, d), jnp.float32),
            pltpu.SemaphoreType.DMA,
        ],
        compiler_params=pltpu.CompilerParams(use_tc_tiling_on_sc=False),
    )
    def gather_kernel(idx_hbm, table_hbm, out_hbm, idx_v, rows_v, gsem):
        wid = lax.axis_index("s") * _NC + lax.axis_index("c")
        base = wid * (n // _NW)
        pltpu.sync_copy(idx_hbm.at[wid], idx_v)

        def body(i, carry):
            pltpu.async_copy(table_hbm.at[idx_v.at[i]], rows_v, gsem).wait()
            pltpu.sync_copy(rows_v, out_hbm.at[pl.ds(base + i * _CH, _CH)])
            return carry

        lax.fori_loop(0, n_ch, body, 0)

    return gather_kernel


def kernel(x, embedding):
    b, h = x.shape
    v, d = embedding.shape
    n = b * h
    n_ch = n // (_NW * _CH)
    idx = x.reshape(_NW, n_ch, _CH).astype(jnp.int32)
    out = _build(n, d, n_ch)(idx, embedding)
    return out.reshape(b, h, d)


# speedup vs baseline: 1.1084x; 1.1084x over previous
import functools

import jax
import jax.numpy as jnp
from jax import lax
from jax.experimental import pallas as pl
from jax.experimental.pallas import tpu as pltpu
from jax.experimental.pallas import tpu_sc as plsc

_NC = 2
_NS = 16
_NW = _NC * _NS
_CH = 128
_NBUF = 4


@functools.lru_cache(maxsize=None)
def _build(n, d, n_ch):
    mesh = plsc.VectorSubcoreMesh(core_axis_name="c", subcore_axis_name="s")

    @functools.partial(
        pl.kernel,
        mesh=mesh,
        out_type=jax.ShapeDtypeStruct((n, d), jnp.float32),
        scratch_types=[
            pltpu.VMEM((n_ch, _CH), jnp.int32),
            pltpu.VMEM((_NBUF, _CH, d), jnp.float32),
            pltpu.SemaphoreType.DMA((_NBUF,)),
            pltpu.SemaphoreType.DMA((_NBUF,)),
        ],
        compiler_params=pltpu.CompilerParams(use_tc_tiling_on_sc=False),
    )
    def gather_kernel(idx_hbm, table_hbm, out_hbm, idx_v, rows_v, gsem, osem):
        wid = lax.axis_index("s") * _NC + lax.axis_index("c")
        base = wid * (n // _NW)
        pltpu.sync_copy(idx_hbm.at[wid], idx_v)

        def start_gather(i, b):
            pltpu.async_copy(table_hbm.at[idx_v.at[i]], rows_v.at[b], gsem.at[b])

        def wait_gather(i, b):
            pltpu.make_async_copy(
                table_hbm.at[idx_v.at[i]], rows_v.at[b], gsem.at[b]).wait()

        def start_out(i, b):
            pltpu.async_copy(
                rows_v.at[b], out_hbm.at[pl.ds(base + i * _CH, _CH)], osem.at[b])

        def wait_out(i, b):
            pltpu.make_async_copy(
                rows_v.at[b], out_hbm.at[pl.ds(base + i * _CH, _CH)],
                osem.at[b]).wait()

        start_gather(0, 0)
        start_gather(1, 1)
        for i in range(2):
            wait_gather(i, i)
            start_out(i, i)
            start_gather(i + 2, i + 2)

        def block(p, carry):
            i0 = 2 + p * _NBUF
            for dlt in range(_NBUF):
                i = i0 + dlt
                b = (2 + dlt) % _NBUF
                b2 = dlt % _NBUF
                wait_gather(i, b)
                start_out(i, b)
                wait_out(i - 2, b2)
                start_gather(i + 2, b2)
            return carry

        lax.fori_loop(0, (n_ch - 4) // _NBUF, block, 0)

        for i in range(n_ch - 2, n_ch):
            b = i % _NBUF
            wait_gather(i, b)
            start_out(i, b)
        for i in range(n_ch - 4, n_ch):
            wait_out(i, i % _NBUF)

    return gather_kernel


def kernel(x, embedding):
    b, h = x.shape
    v, d = embedding.shape
    n = b * h
    n_ch = n // (_NW * _CH)
    idx = x.reshape(_NW, n_ch, _CH).astype(jnp.int32)
    out = _build(n, d, n_ch)(idx, embedding)
    return out.reshape(b, h, d)
